# baseline (speedup 1.0000x reference)
import functools

import jax
import jax.numpy as jnp
from jax import lax
from jax.experimental import pallas as pl
from jax.experimental.pallas import tpu as pltpu
from jax.experimental.pallas import tpu_sc as plsc

D = 128
NCORES = 2
NSUB = 16
NW = NCORES * NSUB
CH = 128


def _splat_lane(vec, lane):
    idx = jnp.full((16, 1), lane, jnp.int32)
    dnums = lax.GatherDimensionNumbers(
        offset_dims=(), collapsed_slice_dims=(0,), start_index_map=(0,))
    return lax.gather(vec, idx, dnums, (1,),
                      mode=lax.GatherScatterMode.PROMISE_IN_BOUNDS)


def _sc_sparse_matmul(xr, rows2d, cols2d, vals2d, n_nodes, n_users):
    nch_tot = rows2d.shape[0]
    npt_sum = nch_tot // NSUB
    if npt_sum % 16 == 0:
        seg = npt_sum // 8
        npt0 = 5 * seg
    else:
        seg = npt_sum // 2
        npt0 = seg
    npt1 = npt_sum - npt0
    n_pad = ((n_nodes + NW * 8 - 1) // (NW * 8)) * (NW * 8)
    rpt = n_pad // NSUB

    mesh = plsc.VectorSubcoreMesh(core_axis_name="c", subcore_axis_name="s")

    @functools.partial(
        pl.kernel,
        out_type=jax.ShapeDtypeStruct((NCORES, n_pad, D), jnp.float32),
        mesh=mesh,
        scratch_types=[
            pltpu.VMEM((seg, 1, CH), jnp.int32),
            pltpu.VMEM((seg, 1, CH), jnp.int32),
            pltpu.VMEM((seg, 1, CH), jnp.float32),
            pltpu.VMEM((2, CH, D), jnp.float32),
            pltpu.VMEM_SHARED((n_pad, D), jnp.float32),
            pltpu.SemaphoreType.DMA,
            pltpu.SemaphoreType.DMA,
            pltpu.SemaphoreType.DMA,
            pltpu.SemaphoreType.DMA,
            pltpu.SemaphoreType.DMA,
        ],
    )
    def body(xr_hbm, rows_hbm, cols_hbm, vals_hbm, out_hbm,
             colsall, rowsall, valsall, gbuf, accum,
             semi, semg0, semg1, sems0, sems1):
        cid = lax.axis_index("c")
        sid = lax.axis_index("s")
        semg = (semg0, semg1)
        sems = (sems0, sems1)

        def zrow(r, _):
            for dd in range(D // 16):
                gbuf[0, r, pl.ds(dd * 16, 16)] = jnp.zeros((16,), jnp.float32)
            return 0
        lax.fori_loop(0, CH, zrow, 0)
        for p in range(rpt // CH):
            pltpu.sync_copy(gbuf.at[0],
                            accum.at[pl.ds(sid * rpt + p * CH, CH)])
        plsc.subcore_barrier()

        my_npt = jnp.where(cid == 0, npt0, npt1)
        cbase = jnp.where(cid == 0, sid * npt0, NSUB * npt0 + sid * npt1)

        def drain_scatter(b):
            pltpu.make_async_copy(gbuf.at[b], accum.at[rowsall.at[0, 0]],
                                  sems[b]).wait()

        def segment(s, _):
            @pl.when(s > 0)
            def _():
                drain_scatter(0)
                drain_scatter(1)
            sb = cbase + s * seg
            cps = [
                pltpu.async_copy(rows_hbm.at[pl.ds(sb, seg)], rowsall, semi),
                pltpu.async_copy(cols_hbm.at[pl.ds(sb, seg)], colsall, semi),
                pltpu.async_copy(vals_hbm.at[pl.ds(sb, seg)], valsall, semi),
            ]
            for cp in cps:
                cp.wait()

            def remap(k, _):
                for g in range(CH // 16):
                    sl = pl.ds(g * 16, 16)
                    cc = colsall[k, 0, sl]
                    colsall[k, 0, sl] = jnp.where(
                        cc >= n_users, cc * 2 - (2 * n_users - 1), cc * 2)
                return 0
            lax.fori_loop(0, seg, remap, 0)

            def pair(k2, _):
                gathers = []
                for b in range(2):
                    c = 2 * k2 + b
                    @pl.when(k2 > 0)
                    def _():
                        drain_scatter(b)
                    gathers.append(pltpu.async_copy(
                        xr_hbm.at[colsall.at[c, 0]], gbuf.at[b], semg[b]))
                for b in range(2):
                    c = 2 * k2 + b
                    gathers[b].wait()

                    def grp(g, _):
                        vals16 = valsall[c, 0, pl.ds(g * 16, 16)]
                        for e in range(16):
                            er = g * 16 + e
                            v = _splat_lane(vals16, e)
                            for dd in range(D // 16):
                                sl = pl.ds(dd * 16, 16)
                                gbuf[b, er, sl] = gbuf[b, er, sl] * v
                        return 0
                    lax.fori_loop(0, CH // 16, grp, 0)
                    pltpu.async_copy(gbuf.at[b], accum.at[rowsall.at[c, 0]],
                                     sems[b], add=True)
                return 0

            lax.fori_loop(0, seg // 2, pair, 0)
            return 0

        lax.fori_loop(0, my_npt // seg, segment, 0)
        drain_scatter(0)
        drain_scatter(1)
        plsc.subcore_barrier()

        for p in range(rpt // CH):
            r0 = sid * rpt + p * CH
            pltpu.sync_copy(accum.at[pl.ds(r0, CH)], gbuf.at[0])
            pltpu.sync_copy(gbuf.at[0], out_hbm.at[cid, pl.ds(r0, CH)])

    return body(xr, rows2d, cols2d, vals2d)


def _tc_combine(x, partials, W1, b1, W2, b2, n_users):
    batch = x.shape[0]
    blk = 1000
    nblk = batch // blk
    ioff = n_users // blk

    def body(x_ref, p0u, p1u, p0i, p1i, w1_ref, b1_ref, w2_ref, b2_ref,
             o_ref):
        xb = x_ref[...]
        h = jnp.maximum(
            jnp.dot(xb, w1_ref[...], preferred_element_type=jnp.float32)
            + b1_ref[...], 0.0)
        w = jax.nn.sigmoid(
            jnp.dot(h, w2_ref[...], preferred_element_type=jnp.float32)
            + b2_ref[...])
        ge_u = p0u[0] + p1u[0]
        ge_i = p0i[0] + p1i[0]
        eu = ge_u - xb[:, :D]
        ei = ge_i - xb[:, D:]
        o_ref[...] = jnp.concatenate([w * eu, w * ei], axis=1)

    pspec = lambda ci, boff: pl.BlockSpec((1, blk, D),
                                          lambda i: (ci, i + boff, 0))
    return pl.pallas_call(
        body,
        grid=(nblk,),
        in_specs=[
            pl.BlockSpec((blk, 2 * D), lambda i: (i, 0)),
            pspec(0, 0), pspec(1, 0), pspec(0, ioff), pspec(1, ioff),
            pl.BlockSpec((2 * D, 64), lambda i: (0, 0)),
            pl.BlockSpec((1, 64), lambda i: (0, 0)),
            pl.BlockSpec((64, D), lambda i: (0, 0)),
            pl.BlockSpec((1, D), lambda i: (0, 0)),
        ],
        out_specs=pl.BlockSpec((blk, 2 * D), lambda i: (i, 0)),
        out_shape=jax.ShapeDtypeStruct((batch, 2 * D), jnp.float32),
    )(x, partials, partials, partials, partials, W1, b1.reshape(1, 64),
      W2, b2.reshape(1, D))


def kernel(t, x, adj_rows, adj_cols, adj_vals, W1, b1, W2, b2):
    batch = x.shape[0]
    n_users = batch
    n_nodes = 2 * batch
    e = adj_rows.shape[0]

    quant = NW * CH * 2
    e_pad = ((e + quant - 1) // quant) * quant
    pad = e_pad - e
    if pad:
        rows = jnp.concatenate([adj_rows, jnp.zeros((pad,), adj_rows.dtype)])
        cols = jnp.concatenate([adj_cols, jnp.zeros((pad,), adj_cols.dtype)])
        vals = jnp.concatenate([adj_vals, jnp.zeros((pad,), adj_vals.dtype)])
    else:
        rows, cols, vals = adj_rows, adj_cols, adj_vals

    xr = x.reshape(n_nodes, D)
    partials = _sc_sparse_matmul(
        xr,
        rows.astype(jnp.int32).reshape(-1, 1, CH),
        cols.astype(jnp.int32).reshape(-1, 1, CH),
        vals.reshape(-1, 1, CH),
        n_nodes, n_users)
    return _tc_combine(x, partials, W1, b1, W2, b2, n_users)


# speedup vs baseline: 1.1915x; 1.1915x over previous
import functools

import jax
import jax.numpy as jnp
from jax import lax
from jax.experimental import pallas as pl
from jax.experimental.pallas import tpu as pltpu
from jax.experimental.pallas import tpu_sc as plsc

D = 128
NCORES = 2
NSUB = 16
NW = NCORES * NSUB
CH = 128


def _splat_lane(vec, lane):
    idx = jnp.full((16, 1), lane, jnp.int32)
    dnums = lax.GatherDimensionNumbers(
        offset_dims=(), collapsed_slice_dims=(0,), start_index_map=(0,))
    return lax.gather(vec, idx, dnums, (1,),
                      mode=lax.GatherScatterMode.PROMISE_IN_BOUNDS)


def _sc_sparse_matmul(xr, rows2d, cols2d, vals2d, n_nodes, n_users):
    nch_tot = rows2d.shape[0]
    npt_sum = nch_tot // NSUB
    if npt_sum % 16 == 0:
        seg = npt_sum // 8
        npt0 = 7 * seg
    else:
        seg = npt_sum // 2
        npt0 = seg
    npt1 = npt_sum - npt0
    n_pad = ((n_nodes + NW * 8 - 1) // (NW * 8)) * (NW * 8)
    rpt = n_pad // NSUB

    mesh = plsc.VectorSubcoreMesh(core_axis_name="c", subcore_axis_name="s")

    @functools.partial(
        pl.kernel,
        out_type=jax.ShapeDtypeStruct((NCORES, n_pad, D), jnp.float32),
        mesh=mesh,
        scratch_types=[
            pltpu.VMEM((seg, 1, CH), jnp.int32),
            pltpu.VMEM((seg, 1, CH), jnp.int32),
            pltpu.VMEM((seg, 1, CH), jnp.float32),
            pltpu.VMEM((2, CH, D), jnp.float32),
            pltpu.VMEM_SHARED((n_pad, D), jnp.float32),
            pltpu.SemaphoreType.DMA,
            pltpu.SemaphoreType.DMA,
            pltpu.SemaphoreType.DMA,
            pltpu.SemaphoreType.DMA,
            pltpu.SemaphoreType.DMA,
        ],
    )
    def body(xr_hbm, rows_hbm, cols_hbm, vals_hbm, out_hbm,
             colsall, rowsall, valsall, gbuf, accum,
             semi, semg0, semg1, sems0, sems1):
        cid = lax.axis_index("c")
        sid = lax.axis_index("s")
        semg = (semg0, semg1)
        sems = (sems0, sems1)

        def zrow(r, _):
            for dd in range(D // 16):
                gbuf[0, r, pl.ds(dd * 16, 16)] = jnp.zeros((16,), jnp.float32)
            return 0
        lax.fori_loop(0, CH, zrow, 0)
        for p in range(rpt // CH):
            pltpu.sync_copy(gbuf.at[0],
                            accum.at[pl.ds(sid * rpt + p * CH, CH)])
        plsc.subcore_barrier()

        my_npt = jnp.where(cid == 0, npt0, npt1)
        cbase = jnp.where(cid == 0, sid * npt0, NSUB * npt0 + sid * npt1)

        def drain_scatter(b):
            pltpu.make_async_copy(gbuf.at[b], accum.at[rowsall.at[0, 0]],
                                  sems[b]).wait()

        def segment(s, _):
            @pl.when(s > 0)
            def _():
                drain_scatter(0)
                drain_scatter(1)
            sb = cbase + s * seg
            cps = [
                pltpu.async_copy(rows_hbm.at[pl.ds(sb, seg)], rowsall, semi),
                pltpu.async_copy(cols_hbm.at[pl.ds(sb, seg)], colsall, semi),
                pltpu.async_copy(vals_hbm.at[pl.ds(sb, seg)], valsall, semi),
            ]
            for cp in cps:
                cp.wait()

            def remap(k, _):
                for g in range(CH // 16):
                    sl = pl.ds(g * 16, 16)
                    cc = colsall[k, 0, sl]
                    colsall[k, 0, sl] = jnp.where(
                        cc >= n_users, cc * 2 - (2 * n_users - 1), cc * 2)
                return 0
            lax.fori_loop(0, seg, remap, 0)

            def pair(k2, _):
                gathers = []
                for b in range(2):
                    c = 2 * k2 + b
                    @pl.when(k2 > 0)
                    def _():
                        drain_scatter(b)
                    gathers.append(pltpu.async_copy(
                        xr_hbm.at[colsall.at[c, 0]], gbuf.at[b], semg[b]))
                for b in range(2):
                    c = 2 * k2 + b
                    gathers[b].wait()

                    def grp(g, _):
                        vals16 = valsall[c, 0, pl.ds(g * 16, 16)]
                        for e in range(16):
                            er = g * 16 + e
                            v = _splat_lane(vals16, e)
                            for dd in range(D // 16):
                                sl = pl.ds(dd * 16, 16)
                                gbuf[b, er, sl] = gbuf[b, er, sl] * v
                        return 0
                    lax.fori_loop(0, CH // 16, grp, 0)
                    pltpu.async_copy(gbuf.at[b], accum.at[rowsall.at[c, 0]],
                                     sems[b], add=True)
                return 0

            lax.fori_loop(0, seg // 2, pair, 0)
            return 0

        lax.fori_loop(0, my_npt // seg, segment, 0)
        drain_scatter(0)
        drain_scatter(1)
        plsc.subcore_barrier()

        for p in range(rpt // CH):
            r0 = sid * rpt + p * CH
            pltpu.sync_copy(accum.at[pl.ds(r0, CH)], gbuf.at[0])
            pltpu.sync_copy(gbuf.at[0], out_hbm.at[cid, pl.ds(r0, CH)])

    return body(xr, rows2d, cols2d, vals2d)


def _tc_combine(x, partials, W1, b1, W2, b2, n_users):
    batch = x.shape[0]
    blk = 1000
    nblk = batch // blk
    ioff = n_users // blk

    def body(x_ref, p0u, p1u, p0i, p1i, w1_ref, b1_ref, w2_ref, b2_ref,
             o_ref):
        xb = x_ref[...]
        h = jnp.maximum(
            jnp.dot(xb, w1_ref[...], preferred_element_type=jnp.float32)
            + b1_ref[...], 0.0)
        w = jax.nn.sigmoid(
            jnp.dot(h, w2_ref[...], preferred_element_type=jnp.float32)
            + b2_ref[...])
        ge_u = p0u[0] + p1u[0]
        ge_i = p0i[0] + p1i[0]
        eu = ge_u - xb[:, :D]
        ei = ge_i - xb[:, D:]
        o_ref[...] = jnp.concatenate([w * eu, w * ei], axis=1)

    pspec = lambda ci, boff: pl.BlockSpec((1, blk, D),
                                          lambda i: (ci, i + boff, 0))
    return pl.pallas_call(
        body,
        grid=(nblk,),
        in_specs=[
            pl.BlockSpec((blk, 2 * D), lambda i: (i, 0)),
            pspec(0, 0), pspec(1, 0), pspec(0, ioff), pspec(1, ioff),
            pl.BlockSpec((2 * D, 64), lambda i: (0, 0)),
            pl.BlockSpec((1, 64), lambda i: (0, 0)),
            pl.BlockSpec((64, D), lambda i: (0, 0)),
            pl.BlockSpec((1, D), lambda i: (0, 0)),
        ],
        out_specs=pl.BlockSpec((blk, 2 * D), lambda i: (i, 0)),
        out_shape=jax.ShapeDtypeStruct((batch, 2 * D), jnp.float32),
    )(x, partials, partials, partials, partials, W1, b1.reshape(1, 64),
      W2, b2.reshape(1, D))


def kernel(t, x, adj_rows, adj_cols, adj_vals, W1, b1, W2, b2):
    batch = x.shape[0]
    n_users = batch
    n_nodes = 2 * batch
    e = adj_rows.shape[0]

    quant = NW * CH * 2
    e_pad = ((e + quant - 1) // quant) * quant
    pad = e_pad - e
    if pad:
        rows = jnp.concatenate([adj_rows, jnp.zeros((pad,), adj_rows.dtype)])
        cols = jnp.concatenate([adj_cols, jnp.zeros((pad,), adj_cols.dtype)])
        vals = jnp.concatenate([adj_vals, jnp.zeros((pad,), adj_vals.dtype)])
    else:
        rows, cols, vals = adj_rows, adj_cols, adj_vals

    xr = x.reshape(n_nodes, D)
    partials = _sc_sparse_matmul(
        xr,
        rows.astype(jnp.int32).reshape(-1, 1, CH),
        cols.astype(jnp.int32).reshape(-1, 1, CH),
        vals.reshape(-1, 1, CH),
        n_nodes, n_users)
    return _tc_combine(x, partials, W1, b1, W2, b2, n_users)
